# baseline (speedup 1.0000x reference)
import functools

import jax
import jax.numpy as jnp
from jax import lax
from jax.experimental import pallas as pl
from jax.experimental.pallas import tpu as pltpu
from jax.experimental.pallas import tpu_sc as plsc

_HIDDEN = 64
_TABLE_ROWS = 100001
_BATCH = 16384

_info = plsc.get_sparse_core_info()
_NC, _NS = _info.num_cores, _info.num_subcores
_NW = _NC * _NS
_BPW = _BATCH // _NW

_mesh = plsc.VectorSubcoreMesh(core_axis_name="c", subcore_axis_name="s")


@functools.partial(
    pl.kernel,
    mesh=_mesh,
    out_type=jax.ShapeDtypeStruct((_HIDDEN, _BATCH), jnp.float32),
    scratch_types=[
        pltpu.VMEM((_BPW,), jnp.int32),
        pltpu.VMEM((_BPW, _HIDDEN), jnp.float32),
        pltpu.VMEM((_HIDDEN, _BPW), jnp.float32),
        pltpu.SemaphoreType.DMA,
    ],
    compiler_params=pltpu.CompilerParams(needs_layout_passes=False),
)
def _embed_gather(table_hbm, labels_hbm, out_hbm, idx_s, rows_v, rows_t, sem):
    wid = lax.axis_index("s") * _NC + lax.axis_index("c")
    base = wid * _BPW
    pltpu.sync_copy(labels_hbm.at[pl.ds(base, _BPW)], idx_s)

    def fire(g, carry):
        vec = idx_s[pl.ds(g * 16, 16)]
        for k in range(16):
            r = vec[k]
            pltpu.async_copy(
                table_hbm.at[0, pl.ds(r, 1)],
                rows_v.at[pl.ds(g * 16 + k, 1)],
                sem,
            )
        return carry

    lax.fori_loop(0, _BPW // 16, fire, 0)
    pltpu.make_async_copy(
        table_hbm.at[0, pl.ds(0, _BPW)], rows_v, sem
    ).wait()

    lanes = lax.broadcasted_iota(jnp.int32, (16,), 0)

    def transpose_col(c, carry):
        col = jnp.broadcast_to(c, (16,))
        for b in range(0, _BPW, 16):
            vec = plsc.load_gather(rows_v, [lanes + b, col])
            rows_t[c, pl.ds(b, 16)] = vec
        return carry

    lax.fori_loop(0, _HIDDEN, transpose_col, 0)
    pltpu.sync_copy(rows_t, out_hbm.at[:, pl.ds(base, _BPW)])


def kernel(labels, embedding_table):
    table3 = embedding_table.reshape(1, _TABLE_ROWS, _HIDDEN)
    out_t = _embed_gather(table3, labels.astype(jnp.int32))
    return out_t.T


# speedup vs baseline: 1.2914x; 1.2914x over previous
import functools

import jax
import jax.numpy as jnp
from jax import lax
from jax.experimental import pallas as pl
from jax.experimental.pallas import tpu as pltpu
from jax.experimental.pallas import tpu_sc as plsc

_HIDDEN = 64
_TABLE_ROWS = 100001
_BATCH = 16384

_info = plsc.get_sparse_core_info()
_NC, _NS = _info.num_cores, _info.num_subcores
_NW = _NC * _NS
_BPW = _BATCH // _NW

_mesh = plsc.VectorSubcoreMesh(core_axis_name="c", subcore_axis_name="s")


@functools.partial(
    pl.kernel,
    mesh=_mesh,
    out_type=jax.ShapeDtypeStruct((_BATCH, _HIDDEN), jnp.float32),
    scratch_types=[
        pltpu.VMEM((_BPW,), jnp.int32),
        pltpu.VMEM((_BPW, _HIDDEN), jnp.float32),
        pltpu.SemaphoreType.DMA,
    ],
    compiler_params=pltpu.CompilerParams(disable_bounds_checks=True),
)
def _embed_gather(table_hbm, labels_hbm, out_hbm, idx_s, rows_v, sem):
    wid = lax.axis_index("s") * _NC + lax.axis_index("c")
    base = wid * _BPW
    pltpu.sync_copy(labels_hbm.at[pl.ds(base, _BPW)], idx_s)

    def body(g, carry):
        vec = idx_s[pl.ds(g * 16, 16)]
        for k in range(16):
            r = vec[k]
            pltpu.async_copy(
                table_hbm.at[0, pl.ds(r, 1)],
                rows_v.at[pl.ds(g * 16 + k, 1)],
                sem,
            )
        return carry

    lax.fori_loop(0, _BPW // 16, body, 0)
    pltpu.make_async_copy(
        table_hbm.at[0, pl.ds(0, _BPW)], rows_v, sem
    ).wait()
    pltpu.sync_copy(rows_v, out_hbm.at[pl.ds(base, _BPW)])


def kernel(labels, embedding_table):
    table3 = embedding_table.reshape(1, _TABLE_ROWS, _HIDDEN)
    return _embed_gather(table3, labels.astype(jnp.int32))
